# baseline (speedup 1.0000x reference)
import jax
import jax.numpy as jnp
from jax import lax
from jax.experimental import pallas as pl
from jax.experimental.pallas import tpu as pltpu
from jax.experimental.pallas import tpu_sc as plsc

N = 100000
E = 3200000
B = 256

SB = 125
NROW = E // SB
RPW = NROW // 32
CH = 4
HB = 8
NB = RPW // (2 * HB)
CB = 16
NBC = RPW // (2 * CB)

NP = 102400
RSUB = NP // 16

PB = 100
PR = NP // PB
PRW = PR // 32
PT = 384

GV = NP // 8
RV = 2 * NP // 8
BR = 1600
NBLK = GV // BR

import functools


@functools.lru_cache(maxsize=1)
def _sc_mesh():
    return plsc.VectorSubcoreMesh(core_axis_name="c", subcore_axis_name="s")


def _fill(ref, nrows, value):
    def body(i, _):
        ref[i, :] = jnp.full((16,), value, jnp.float32)
        return 0
    lax.fori_loop(0, nrows, body, 0)


def _zero_table(S, z_hbm, sid, rows_per_sub):
    pltpu.sync_copy(z_hbm.at[pl.ds(sid * rows_per_sub, rows_per_sub)],
                    S.at[pl.ds(sid * rows_per_sub, rows_per_sub)])


def _count_body(z_hbm, ones_hbm, dst_hbm, out_hbm, idst_a, idst_b, ones_buf, S,
                sem_s, sem_ia, sem_ib):
    cid = lax.axis_index("c")
    sid = lax.axis_index("s")
    pltpu.sync_copy(ones_hbm, ones_buf)
    _zero_table(S, z_hbm, sid, RSUB)
    plsc.subcore_barrier()
    base = cid * (16 * RPW) + sid * RPW
    pltpu.async_copy(dst_hbm.at[pl.ds(base, CB)], idst_a, sem_ia)
    pltpu.async_copy(dst_hbm.at[pl.ds(base + CB, CB)], idst_b, sem_ib)

    def it_body(k, _):
        rn = jnp.minimum(base + (k + 1) * 2 * CB, NROW - 2 * CB)
        pltpu.make_async_copy(dst_hbm.at[pl.ds(0, CB)], idst_a, sem_ia).wait()
        da = [pltpu.async_copy(ones_buf, S.at[idst_a.at[j]], sem_s, add=True)
              for j in range(CB)]
        pltpu.make_async_copy(dst_hbm.at[pl.ds(0, CB)], idst_b, sem_ib).wait()
        db = [pltpu.async_copy(ones_buf, S.at[idst_b.at[j]], sem_s, add=True)
              for j in range(CB)]
        for d in da:
            d.wait()
        pltpu.async_copy(dst_hbm.at[pl.ds(rn, CB)], idst_a, sem_ia)
        for d in db:
            d.wait()
        rn2 = jnp.minimum(rn + CB, NROW - CB)
        pltpu.async_copy(dst_hbm.at[pl.ds(rn2, CB)], idst_b, sem_ib)
        return 0
    lax.fori_loop(0, NBC, it_body, 0)
    pltpu.make_async_copy(dst_hbm.at[pl.ds(0, CB)], idst_a, sem_ia).wait()
    pltpu.make_async_copy(dst_hbm.at[pl.ds(0, CB)], idst_b, sem_ib).wait()
    plsc.subcore_barrier()
    pltpu.sync_copy(S.at[pl.ds(sid * RSUB, RSUB)],
                    out_hbm.at[pl.ds(cid * NP + sid * RSUB, RSUB)])


@functools.lru_cache(maxsize=1)
def _sc_count():
    return pl.kernel(
        _count_body,
        out_type=jax.ShapeDtypeStruct((2 * NP, 8), jnp.float32),
        mesh=_sc_mesh(),
        compiler_params=pltpu.CompilerParams(use_tc_tiling_on_sc=False),
        scratch_types=[
            pltpu.VMEM((CB, SB), jnp.int32),
            pltpu.VMEM((CB, SB), jnp.int32),
            pltpu.VMEM((SB, 8), jnp.float32),
            pltpu.VMEM_SHARED((NP, 8), jnp.float32),
            pltpu.SemaphoreType.DMA,
            pltpu.SemaphoreType.DMA,
            pltpu.SemaphoreType.DMA,
        ],
    )


def _layer_body(z_hbm, g_hbm, src_hbm, dst_hbm, out_hbm,
                isrc_a, idst_a, isrc_b, idst_b, rows_buf, S,
                sem_g, sem_s, sem_ia, sem_ib):
    cid = lax.axis_index("c")
    sid = lax.axis_index("s")
    _zero_table(S, z_hbm, sid, RSUB)
    plsc.subcore_barrier()
    base = cid * (16 * RPW) + sid * RPW
    pltpu.async_copy(src_hbm.at[pl.ds(base, HB)], isrc_a, sem_ia)
    pltpu.async_copy(dst_hbm.at[pl.ds(base, HB)], idst_a, sem_ia)
    pltpu.async_copy(src_hbm.at[pl.ds(base + HB, HB)], isrc_b, sem_ib)
    pltpu.async_copy(dst_hbm.at[pl.ds(base + HB, HB)], idst_b, sem_ib)

    def _gfire(isrc, g0, rb):
        return [pltpu.async_copy(g_hbm.at[isrc.at[CH * g0 + j]],
                                 rows_buf.at[rb, j], sem_g)
                for j in range(CH)]

    def _sfire(idst, g0, rb):
        return [pltpu.async_copy(rows_buf.at[rb, j],
                                 S.at[idst.at[CH * g0 + j]], sem_s, add=True)
                for j in range(CH)]

    def _waitall(descs):
        for d in descs:
            d.wait()

    def it_body(k, _):
        rn = jnp.minimum(base + (k + 1) * 2 * HB, NROW - 2 * HB)
        pltpu.make_async_copy(src_hbm.at[pl.ds(0, HB)], isrc_a, sem_ia).wait()
        pltpu.make_async_copy(src_hbm.at[pl.ds(0, HB)], idst_a, sem_ia).wait()
        g0 = _gfire(isrc_a, 0, 0)
        g1 = _gfire(isrc_a, 1, 1)
        pltpu.make_async_copy(src_hbm.at[pl.ds(0, HB)], isrc_b, sem_ib).wait()
        pltpu.make_async_copy(src_hbm.at[pl.ds(0, HB)], idst_b, sem_ib).wait()
        g2 = _gfire(isrc_b, 0, 2)
        _waitall(g0)
        s0 = _sfire(idst_a, 0, 0)
        _waitall(g1)
        s1 = _sfire(idst_a, 1, 1)
        _waitall(s0)
        g3 = _gfire(isrc_b, 1, 0)
        _waitall(g2)
        s2 = _sfire(idst_b, 0, 2)
        _waitall(s1)
        pltpu.async_copy(src_hbm.at[pl.ds(rn, HB)], isrc_a, sem_ia)
        pltpu.async_copy(dst_hbm.at[pl.ds(rn, HB)], idst_a, sem_ia)
        _waitall(g3)
        s3 = _sfire(idst_b, 1, 0)
        _waitall(s2)
        _waitall(s3)
        rn2 = jnp.minimum(rn + HB, NROW - HB)
        pltpu.async_copy(src_hbm.at[pl.ds(rn2, HB)], isrc_b, sem_ib)
        pltpu.async_copy(dst_hbm.at[pl.ds(rn2, HB)], idst_b, sem_ib)
        return 0
    lax.fori_loop(0, NB, it_body, 0)
    pltpu.make_async_copy(src_hbm.at[pl.ds(0, HB)], isrc_a, sem_ia).wait()
    pltpu.make_async_copy(src_hbm.at[pl.ds(0, HB)], idst_a, sem_ia).wait()
    pltpu.make_async_copy(src_hbm.at[pl.ds(0, HB)], isrc_b, sem_ib).wait()
    pltpu.make_async_copy(src_hbm.at[pl.ds(0, HB)], idst_b, sem_ib).wait()
    plsc.subcore_barrier()
    pltpu.sync_copy(S.at[pl.ds(sid * RSUB, RSUB)],
                    out_hbm.at[pl.ds(cid * NP + sid * RSUB, RSUB)])


@functools.lru_cache(maxsize=1)
def _sc_layer():
    return pl.kernel(
        _layer_body,
        out_type=jax.ShapeDtypeStruct((2 * NP, 16), jnp.float32),
        mesh=_sc_mesh(),
        compiler_params=pltpu.CompilerParams(use_tc_tiling_on_sc=False),
        scratch_types=[
            pltpu.VMEM((HB, SB), jnp.int32),
            pltpu.VMEM((HB, SB), jnp.int32),
            pltpu.VMEM((HB, SB), jnp.int32),
            pltpu.VMEM((HB, SB), jnp.int32),
            pltpu.VMEM((3, CH, SB, 16), jnp.float32),
            pltpu.VMEM_SHARED((NP, 16), jnp.float32),
            pltpu.SemaphoreType.DMA,
            pltpu.SemaphoreType.DMA,
            pltpu.SemaphoreType.DMA,
            pltpu.SemaphoreType.DMA,
        ],
    )


def _pool_body(z_hbm, h_hbm, b_hbm, sum_hbm, cnt_hbm,
               b_buf, rows_buf, ones_buf, P, C):
    cid = lax.axis_index("c")
    sid = lax.axis_index("s")
    _fill(ones_buf, PB, 1.0)
    _zero_table(P, z_hbm, sid, PT // 16)
    _zero_table(C, z_hbm, sid, PT // 16)
    plsc.subcore_barrier()
    base = cid * (16 * PRW) + sid * PRW
    pltpu.sync_copy(b_hbm.at[pl.ds(base, PRW)], b_buf)
    pltpu.sync_copy(h_hbm.at[pl.ds(base * PB, PRW * PB)], rows_buf)

    def jloop(j, _):
        pltpu.sync_copy(rows_buf.at[pl.ds(j * PB, PB)], P.at[b_buf.at[j]], add=True)
        pltpu.sync_copy(ones_buf, C.at[b_buf.at[j]], add=True)
        return 0
    lax.fori_loop(0, PRW, jloop, 0)
    plsc.subcore_barrier()
    nsub = B // 16
    pltpu.sync_copy(P.at[pl.ds(sid * nsub, nsub)],
                    sum_hbm.at[pl.ds(cid * B + sid * nsub, nsub)])
    pltpu.sync_copy(C.at[pl.ds(sid * nsub, nsub)],
                    cnt_hbm.at[pl.ds(cid * B + sid * nsub, nsub)])


@functools.lru_cache(maxsize=1)
def _sc_pool():
    return pl.kernel(
        _pool_body,
        out_type=[jax.ShapeDtypeStruct((2 * B, 16), jnp.float32),
                  jax.ShapeDtypeStruct((2 * B, 16), jnp.float32)],
        mesh=_sc_mesh(),
        compiler_params=pltpu.CompilerParams(use_tc_tiling_on_sc=False),
        scratch_types=[
            pltpu.VMEM((PRW, PB), jnp.int32),
            pltpu.VMEM((PRW * PB, 16), jnp.float32),
            pltpu.VMEM((PB, 16), jnp.float32),
            pltpu.VMEM_SHARED((PT, 16), jnp.float32),
            pltpu.VMEM_SHARED((PT, 16), jnp.float32),
        ],
    )


def _tc1_body(c0, c1, xv, bd, dinv_ref, g1_ref):
    dinv = lax.rsqrt(c0[...] + c1[...] + 1.0)
    dinv_ref[...] = dinv
    g1_ref[...] = dinv * jnp.dot(xv[...], bd[...],
                                 preferred_element_type=jnp.float32)


_tc1 = pl.pallas_call(
    _tc1_body,
    grid=(NBLK,),
    in_specs=[pl.BlockSpec((BR, 128), lambda i: (i, 0)),
              pl.BlockSpec((BR, 128), lambda i: (i, 0)),
              pl.BlockSpec((BR, 128), lambda i: (i, 0)),
              pl.BlockSpec((128, 128), lambda i: (0, 0))],
    out_specs=[pl.BlockSpec((BR, 128), lambda i: (i, 0)),
               pl.BlockSpec((BR, 128), lambda i: (i, 0))],
    out_shape=[jax.ShapeDtypeStruct((GV, 128), jnp.float32),
               jax.ShapeDtypeStruct((GV, 128), jnp.float32)],
)


def _tcmid_body(s0, s1, g, dinv, bt, bd, gout):
    h = dinv[...] * (s0[...] + s1[...] + g[...]) + bt[...]
    gout[...] = dinv[...] * jnp.dot(h, bd[...],
                                    preferred_element_type=jnp.float32)


_tc_mid = pl.pallas_call(
    _tcmid_body,
    grid=(NBLK,),
    in_specs=[pl.BlockSpec((BR, 128), lambda i: (i, 0)),
              pl.BlockSpec((BR, 128), lambda i: (i + NBLK, 0)),
              pl.BlockSpec((BR, 128), lambda i: (i, 0)),
              pl.BlockSpec((BR, 128), lambda i: (i, 0)),
              pl.BlockSpec((1, 128), lambda i: (0, 0)),
              pl.BlockSpec((128, 128), lambda i: (0, 0))],
    out_specs=pl.BlockSpec((BR, 128), lambda i: (i, 0)),
    out_shape=jax.ShapeDtypeStruct((GV, 128), jnp.float32),
)


def _tclast_body(s0, s1, g, dinv, bt, hout):
    hout[...] = dinv[...] * (s0[...] + s1[...] + g[...]) + bt[...]


_tc_last = pl.pallas_call(
    _tclast_body,
    grid=(NBLK,),
    in_specs=[pl.BlockSpec((BR, 128), lambda i: (i, 0)),
              pl.BlockSpec((BR, 128), lambda i: (i + NBLK, 0)),
              pl.BlockSpec((BR, 128), lambda i: (i, 0)),
              pl.BlockSpec((BR, 128), lambda i: (i, 0)),
              pl.BlockSpec((1, 128), lambda i: (0, 0))],
    out_specs=pl.BlockSpec((BR, 128), lambda i: (i, 0)),
    out_shape=jax.ShapeDtypeStruct((GV, 128), jnp.float32),
)


def _tcpool_body(p0, p1, c0, c1, wl, blt, z):
    pooled = (p0[...] + p1[...]) / jnp.maximum(c0[...] + c1[...], 1.0)
    z[...] = jax.nn.sigmoid(
        jnp.dot(pooled, wl[...], preferred_element_type=jnp.float32) + blt[...])


_tc_pool = pl.pallas_call(
    _tcpool_body,
    grid=(1,),
    in_specs=[pl.BlockSpec((B // 8, 128), lambda i: (0, 0)),
              pl.BlockSpec((B // 8, 128), lambda i: (1, 0)),
              pl.BlockSpec((B // 8, 128), lambda i: (0, 0)),
              pl.BlockSpec((B // 8, 128), lambda i: (1, 0)),
              pl.BlockSpec((128, 128), lambda i: (0, 0)),
              pl.BlockSpec((1, 128), lambda i: (0, 0))],
    out_specs=pl.BlockSpec((B // 8, 128), lambda i: (0, 0)),
    out_shape=jax.ShapeDtypeStruct((B // 8, 128), jnp.float32),
)


def kernel(x, edge_index, batch, W1, b1, W2, b2, W3, b3, Wl, bl):
    f32 = jnp.float32
    src2d = edge_index[0].reshape(NROW, SB)
    dst2d = edge_index[1].reshape(NROW, SB)
    batch2d = jnp.pad(batch, (0, NP - N), constant_values=B).reshape(PR, PB)
    xpad = jnp.pad(x, ((0, NP - N), (0, 12)))
    eye8 = jnp.eye(8, dtype=f32)
    bd1 = jnp.kron(eye8, jnp.pad(W1, ((0, 12), (0, 0))))
    bd2 = jnp.kron(eye8, W2)
    bd3 = jnp.kron(eye8, W3)
    bdl = jnp.kron(eye8, jnp.pad(Wl, ((0, 0), (0, 15))))
    bt1 = jnp.tile(b1, 8).reshape(1, 128)
    bt2 = jnp.tile(b2, 8).reshape(1, 128)
    bt3 = jnp.tile(b3, 8).reshape(1, 128)
    btl = jnp.tile(jnp.pad(bl, (0, 15)), 8).reshape(1, 128)

    zeros = jnp.zeros((NP, 16), f32)
    ones8 = jnp.ones((SB, 8), f32)
    sc_count, sc_layer, sc_pool = _sc_count(), _sc_layer(), _sc_pool()
    cnt8 = sc_count(zeros.reshape(2 * NP, 8), ones8, dst2d)
    c0v = jnp.repeat(cnt8[:NP, :1], 16, axis=1).reshape(GV, 128)
    c1v = jnp.repeat(cnt8[NP:, :1], 16, axis=1).reshape(GV, 128)
    dinv_v, g1v = _tc1(c0v, c1v, xpad.reshape(GV, 128), bd1)
    s1 = sc_layer(zeros, g1v.reshape(NP, 16), src2d, dst2d).reshape(RV, 128)
    g2v = _tc_mid(s1, s1, g1v, dinv_v, bt1, bd2)
    s2 = sc_layer(zeros, g2v.reshape(NP, 16), src2d, dst2d).reshape(RV, 128)
    g3v = _tc_mid(s2, s2, g2v, dinv_v, bt2, bd3)
    s3 = sc_layer(zeros, g3v.reshape(NP, 16), src2d, dst2d).reshape(RV, 128)
    h3v = _tc_last(s3, s3, g3v, dinv_v, bt3)
    psum, pcnt = sc_pool(zeros, h3v.reshape(NP, 16), batch2d)
    zv = _tc_pool(psum.reshape(2 * B // 8, 128), psum.reshape(2 * B // 8, 128),
                  pcnt.reshape(2 * B // 8, 128), pcnt.reshape(2 * B // 8, 128),
                  bdl, btl)
    return zv.reshape(B, 16)[:, :1]


# speedup vs baseline: 1.2305x; 1.2305x over previous
import jax
import jax.numpy as jnp
from jax import lax
from jax.experimental import pallas as pl
from jax.experimental.pallas import tpu as pltpu
from jax.experimental.pallas import tpu_sc as plsc

N = 100000
E = 3200000
B = 256

SB = 125
NROW = E // SB
RPW = NROW // 32
CH = 4
HB = 8
NB = RPW // (2 * HB)
CB = 16
NBC = RPW // (2 * CB)

NP = 102400
RSUB = NP // 16

PB = 100
PR = NP // PB
PRW = PR // 32
PT = 384

GV = NP // 8
RV = 2 * NP // 8
BR = 1600
NBLK = GV // BR

import functools


@functools.lru_cache(maxsize=1)
def _sc_mesh():
    return plsc.VectorSubcoreMesh(core_axis_name="c", subcore_axis_name="s")


def _fill(ref, nrows, value):
    def body(i, _):
        ref[i, :] = jnp.full((16,), value, jnp.float32)
        return 0
    lax.fori_loop(0, nrows, body, 0)


def _zero_table(S, z_hbm, sid, rows_per_sub):
    pltpu.sync_copy(z_hbm.at[pl.ds(sid * rows_per_sub, rows_per_sub)],
                    S.at[pl.ds(sid * rows_per_sub, rows_per_sub)])


def _count_body(z_hbm, dst_hbm, out_hbm, idst_a, idst_b, ones_buf, S,
                sem_s, sem_ia, sem_ib):
    cid = lax.axis_index("c")
    sid = lax.axis_index("s")
    _fill(ones_buf, SB, 1.0)
    _zero_table(S, z_hbm, sid, RSUB)
    plsc.subcore_barrier()
    base = cid * (16 * RPW) + sid * RPW
    pltpu.async_copy(dst_hbm.at[pl.ds(base, CB)], idst_a, sem_ia)
    pltpu.async_copy(dst_hbm.at[pl.ds(base + CB, CB)], idst_b, sem_ib)

    def it_body(k, _):
        rn = jnp.minimum(base + (k + 1) * 2 * CB, NROW - 2 * CB)
        pltpu.make_async_copy(dst_hbm.at[pl.ds(0, CB)], idst_a, sem_ia).wait()
        da = [pltpu.async_copy(ones_buf, S.at[idst_a.at[j]], sem_s, add=True)
              for j in range(CB)]
        pltpu.make_async_copy(dst_hbm.at[pl.ds(0, CB)], idst_b, sem_ib).wait()
        db = [pltpu.async_copy(ones_buf, S.at[idst_b.at[j]], sem_s, add=True)
              for j in range(CB)]
        for d in da:
            d.wait()
        pltpu.async_copy(dst_hbm.at[pl.ds(rn, CB)], idst_a, sem_ia)
        for d in db:
            d.wait()
        rn2 = jnp.minimum(rn + CB, NROW - CB)
        pltpu.async_copy(dst_hbm.at[pl.ds(rn2, CB)], idst_b, sem_ib)
        return 0
    lax.fori_loop(0, NBC, it_body, 0)
    pltpu.make_async_copy(dst_hbm.at[pl.ds(0, CB)], idst_a, sem_ia).wait()
    pltpu.make_async_copy(dst_hbm.at[pl.ds(0, CB)], idst_b, sem_ib).wait()
    plsc.subcore_barrier()
    pltpu.sync_copy(S.at[pl.ds(sid * RSUB, RSUB)],
                    out_hbm.at[pl.ds(cid * NP + sid * RSUB, RSUB)])


@functools.lru_cache(maxsize=1)
def _sc_count():
    return pl.kernel(
        _count_body,
        out_type=jax.ShapeDtypeStruct((2 * NP, 16), jnp.float32),
        mesh=_sc_mesh(),
        compiler_params=pltpu.CompilerParams(use_tc_tiling_on_sc=False),
        scratch_types=[
            pltpu.VMEM((CB, SB), jnp.int32),
            pltpu.VMEM((CB, SB), jnp.int32),
            pltpu.VMEM((SB, 16), jnp.float32),
            pltpu.VMEM_SHARED((NP, 16), jnp.float32),
            pltpu.SemaphoreType.DMA,
            pltpu.SemaphoreType.DMA,
            pltpu.SemaphoreType.DMA,
        ],
    )


def _layer_body(z_hbm, g_hbm, src_hbm, dst_hbm, out_hbm,
                isrc_a, idst_a, isrc_b, idst_b, rows_buf, S,
                sem_g, sem_s, sem_ia, sem_ib):
    cid = lax.axis_index("c")
    sid = lax.axis_index("s")
    _zero_table(S, z_hbm, sid, RSUB)
    plsc.subcore_barrier()
    base = cid * (16 * RPW) + sid * RPW
    pltpu.async_copy(src_hbm.at[pl.ds(base, HB)], isrc_a, sem_ia)
    pltpu.async_copy(dst_hbm.at[pl.ds(base, HB)], idst_a, sem_ia)
    pltpu.async_copy(src_hbm.at[pl.ds(base + HB, HB)], isrc_b, sem_ib)
    pltpu.async_copy(dst_hbm.at[pl.ds(base + HB, HB)], idst_b, sem_ib)

    def _gfire(isrc, g0, rb):
        return [pltpu.async_copy(g_hbm.at[isrc.at[CH * g0 + j]],
                                 rows_buf.at[rb, j], sem_g)
                for j in range(CH)]

    def _sfire(idst, g0, rb):
        return [pltpu.async_copy(rows_buf.at[rb, j],
                                 S.at[idst.at[CH * g0 + j]], sem_s, add=True)
                for j in range(CH)]

    def _waitall(descs):
        for d in descs:
            d.wait()

    def it_body(k, _):
        rn = jnp.minimum(base + (k + 1) * 2 * HB, NROW - 2 * HB)
        pltpu.make_async_copy(src_hbm.at[pl.ds(0, HB)], isrc_a, sem_ia).wait()
        pltpu.make_async_copy(src_hbm.at[pl.ds(0, HB)], idst_a, sem_ia).wait()
        g0 = _gfire(isrc_a, 0, 0)
        g1 = _gfire(isrc_a, 1, 1)
        pltpu.make_async_copy(src_hbm.at[pl.ds(0, HB)], isrc_b, sem_ib).wait()
        pltpu.make_async_copy(src_hbm.at[pl.ds(0, HB)], idst_b, sem_ib).wait()
        g2 = _gfire(isrc_b, 0, 2)
        _waitall(g0)
        s0 = _sfire(idst_a, 0, 0)
        _waitall(g1)
        s1 = _sfire(idst_a, 1, 1)
        _waitall(s0)
        g3 = _gfire(isrc_b, 1, 0)
        _waitall(g2)
        s2 = _sfire(idst_b, 0, 2)
        _waitall(s1)
        pltpu.async_copy(src_hbm.at[pl.ds(rn, HB)], isrc_a, sem_ia)
        pltpu.async_copy(dst_hbm.at[pl.ds(rn, HB)], idst_a, sem_ia)
        _waitall(g3)
        s3 = _sfire(idst_b, 1, 0)
        _waitall(s2)
        _waitall(s3)
        rn2 = jnp.minimum(rn + HB, NROW - HB)
        pltpu.async_copy(src_hbm.at[pl.ds(rn2, HB)], isrc_b, sem_ib)
        pltpu.async_copy(dst_hbm.at[pl.ds(rn2, HB)], idst_b, sem_ib)
        return 0
    lax.fori_loop(0, NB, it_body, 0)
    pltpu.make_async_copy(src_hbm.at[pl.ds(0, HB)], isrc_a, sem_ia).wait()
    pltpu.make_async_copy(src_hbm.at[pl.ds(0, HB)], idst_a, sem_ia).wait()
    pltpu.make_async_copy(src_hbm.at[pl.ds(0, HB)], isrc_b, sem_ib).wait()
    pltpu.make_async_copy(src_hbm.at[pl.ds(0, HB)], idst_b, sem_ib).wait()
    plsc.subcore_barrier()
    pltpu.sync_copy(S.at[pl.ds(sid * RSUB, RSUB)],
                    out_hbm.at[pl.ds(cid * NP + sid * RSUB, RSUB)])


@functools.lru_cache(maxsize=1)
def _sc_layer():
    return pl.kernel(
        _layer_body,
        out_type=jax.ShapeDtypeStruct((2 * NP, 16), jnp.float32),
        mesh=_sc_mesh(),
        compiler_params=pltpu.CompilerParams(use_tc_tiling_on_sc=False),
        scratch_types=[
            pltpu.VMEM((HB, SB), jnp.int32),
            pltpu.VMEM((HB, SB), jnp.int32),
            pltpu.VMEM((HB, SB), jnp.int32),
            pltpu.VMEM((HB, SB), jnp.int32),
            pltpu.VMEM((3, CH, SB, 16), jnp.float32),
            pltpu.VMEM_SHARED((NP, 16), jnp.float32),
            pltpu.SemaphoreType.DMA,
            pltpu.SemaphoreType.DMA,
            pltpu.SemaphoreType.DMA,
            pltpu.SemaphoreType.DMA,
        ],
    )


def _pool_body(z_hbm, h_hbm, b_hbm, sum_hbm, cnt_hbm,
               b_buf, rows_buf, ones_buf, P, C):
    cid = lax.axis_index("c")
    sid = lax.axis_index("s")
    _fill(ones_buf, PB, 1.0)
    _zero_table(P, z_hbm, sid, PT // 16)
    _zero_table(C, z_hbm, sid, PT // 16)
    plsc.subcore_barrier()
    base = cid * (16 * PRW) + sid * PRW
    pltpu.sync_copy(b_hbm.at[pl.ds(base, PRW)], b_buf)
    pltpu.sync_copy(h_hbm.at[pl.ds(base * PB, PRW * PB)], rows_buf)

    def jloop(j, _):
        pltpu.sync_copy(rows_buf.at[pl.ds(j * PB, PB)], P.at[b_buf.at[j]], add=True)
        pltpu.sync_copy(ones_buf, C.at[b_buf.at[j]], add=True)
        return 0
    lax.fori_loop(0, PRW, jloop, 0)
    plsc.subcore_barrier()
    nsub = B // 16
    pltpu.sync_copy(P.at[pl.ds(sid * nsub, nsub)],
                    sum_hbm.at[pl.ds(cid * B + sid * nsub, nsub)])
    pltpu.sync_copy(C.at[pl.ds(sid * nsub, nsub)],
                    cnt_hbm.at[pl.ds(cid * B + sid * nsub, nsub)])


@functools.lru_cache(maxsize=1)
def _sc_pool():
    return pl.kernel(
        _pool_body,
        out_type=[jax.ShapeDtypeStruct((2 * B, 16), jnp.float32),
                  jax.ShapeDtypeStruct((2 * B, 16), jnp.float32)],
        mesh=_sc_mesh(),
        compiler_params=pltpu.CompilerParams(use_tc_tiling_on_sc=False),
        scratch_types=[
            pltpu.VMEM((PRW, PB), jnp.int32),
            pltpu.VMEM((PRW * PB, 16), jnp.float32),
            pltpu.VMEM((PB, 16), jnp.float32),
            pltpu.VMEM_SHARED((PT, 16), jnp.float32),
            pltpu.VMEM_SHARED((PT, 16), jnp.float32),
        ],
    )


def _tc0_body(xv, bd, h1_ref):
    h1_ref[...] = jnp.dot(xv[...], bd[...], preferred_element_type=jnp.float32)


_tc0 = pl.pallas_call(
    _tc0_body,
    grid=(NBLK,),
    in_specs=[pl.BlockSpec((BR, 128), lambda i: (i, 0)),
              pl.BlockSpec((128, 128), lambda i: (0, 0))],
    out_specs=pl.BlockSpec((BR, 128), lambda i: (i, 0)),
    out_shape=jax.ShapeDtypeStruct((GV, 128), jnp.float32),
)


def _tc1_body(c0, c1, h1, dinv_ref, g1_ref):
    dinv = lax.rsqrt(c0[...] + c1[...] + 1.0)
    dinv_ref[...] = dinv
    g1_ref[...] = dinv * h1[...]


_tc1 = pl.pallas_call(
    _tc1_body,
    grid=(NBLK,),
    in_specs=[pl.BlockSpec((BR, 128), lambda i: (i, 0)),
              pl.BlockSpec((BR, 128), lambda i: (i + NBLK, 0)),
              pl.BlockSpec((BR, 128), lambda i: (i, 0))],
    out_specs=[pl.BlockSpec((BR, 128), lambda i: (i, 0)),
               pl.BlockSpec((BR, 128), lambda i: (i, 0))],
    out_shape=[jax.ShapeDtypeStruct((GV, 128), jnp.float32),
               jax.ShapeDtypeStruct((GV, 128), jnp.float32)],
)


def _tcmid_body(s0, s1, g, dinv, bt, bd, gout):
    h = dinv[...] * (s0[...] + s1[...] + g[...]) + bt[...]
    gout[...] = dinv[...] * jnp.dot(h, bd[...],
                                    preferred_element_type=jnp.float32)


_tc_mid = pl.pallas_call(
    _tcmid_body,
    grid=(NBLK,),
    in_specs=[pl.BlockSpec((BR, 128), lambda i: (i, 0)),
              pl.BlockSpec((BR, 128), lambda i: (i + NBLK, 0)),
              pl.BlockSpec((BR, 128), lambda i: (i, 0)),
              pl.BlockSpec((BR, 128), lambda i: (i, 0)),
              pl.BlockSpec((1, 128), lambda i: (0, 0)),
              pl.BlockSpec((128, 128), lambda i: (0, 0))],
    out_specs=pl.BlockSpec((BR, 128), lambda i: (i, 0)),
    out_shape=jax.ShapeDtypeStruct((GV, 128), jnp.float32),
)


def _tclast_body(s0, s1, g, dinv, bt, hout):
    hout[...] = dinv[...] * (s0[...] + s1[...] + g[...]) + bt[...]


_tc_last = pl.pallas_call(
    _tclast_body,
    grid=(NBLK,),
    in_specs=[pl.BlockSpec((BR, 128), lambda i: (i, 0)),
              pl.BlockSpec((BR, 128), lambda i: (i + NBLK, 0)),
              pl.BlockSpec((BR, 128), lambda i: (i, 0)),
              pl.BlockSpec((BR, 128), lambda i: (i, 0)),
              pl.BlockSpec((1, 128), lambda i: (0, 0))],
    out_specs=pl.BlockSpec((BR, 128), lambda i: (i, 0)),
    out_shape=jax.ShapeDtypeStruct((GV, 128), jnp.float32),
)


def _tcpool_body(p0, p1, c0, c1, wl, blt, z):
    pooled = (p0[...] + p1[...]) / jnp.maximum(c0[...] + c1[...], 1.0)
    z[...] = jax.nn.sigmoid(
        jnp.dot(pooled, wl[...], preferred_element_type=jnp.float32) + blt[...])


_tc_pool = pl.pallas_call(
    _tcpool_body,
    grid=(1,),
    in_specs=[pl.BlockSpec((B // 8, 128), lambda i: (0, 0)),
              pl.BlockSpec((B // 8, 128), lambda i: (1, 0)),
              pl.BlockSpec((B // 8, 128), lambda i: (0, 0)),
              pl.BlockSpec((B // 8, 128), lambda i: (1, 0)),
              pl.BlockSpec((128, 128), lambda i: (0, 0)),
              pl.BlockSpec((1, 128), lambda i: (0, 0))],
    out_specs=pl.BlockSpec((B // 8, 128), lambda i: (0, 0)),
    out_shape=jax.ShapeDtypeStruct((B // 8, 128), jnp.float32),
)


def kernel(x, edge_index, batch, W1, b1, W2, b2, W3, b3, Wl, bl):
    f32 = jnp.float32
    src2d = edge_index[0].reshape(NROW, SB)
    dst2d = edge_index[1].reshape(NROW, SB)
    batch2d = jnp.pad(batch, (0, NP - N), constant_values=B).reshape(PR, PB)
    xpad = jnp.pad(x, ((0, NP - N), (0, 12)))
    eye8 = jnp.eye(8, dtype=f32)
    bd1 = jnp.kron(eye8, jnp.pad(W1, ((0, 12), (0, 0))))
    bd2 = jnp.kron(eye8, W2)
    bd3 = jnp.kron(eye8, W3)
    bdl = jnp.kron(eye8, jnp.pad(Wl, ((0, 0), (0, 15))))
    bt1 = jnp.tile(b1, 8).reshape(1, 128)
    bt2 = jnp.tile(b2, 8).reshape(1, 128)
    bt3 = jnp.tile(b3, 8).reshape(1, 128)
    btl = jnp.tile(jnp.pad(bl, (0, 15)), 8).reshape(1, 128)

    zeros = jnp.zeros((NP, 16), f32)
    sc_count, sc_layer, sc_pool = _sc_count(), _sc_layer(), _sc_pool()
    h1v = _tc0(xpad.reshape(GV, 128), bd1)
    cnt = sc_count(zeros, dst2d).reshape(RV, 128)
    dinv_v, g1v = _tc1(cnt, cnt, h1v)
    s1 = sc_layer(zeros, g1v.reshape(NP, 16), src2d, dst2d).reshape(RV, 128)
    g2v = _tc_mid(s1, s1, g1v, dinv_v, bt1, bd2)
    s2 = sc_layer(zeros, g2v.reshape(NP, 16), src2d, dst2d).reshape(RV, 128)
    g3v = _tc_mid(s2, s2, g2v, dinv_v, bt2, bd3)
    s3 = sc_layer(zeros, g3v.reshape(NP, 16), src2d, dst2d).reshape(RV, 128)
    h3v = _tc_last(s3, s3, g3v, dinv_v, bt3)
    psum, pcnt = sc_pool(zeros, h3v.reshape(NP, 16), batch2d)
    zv = _tc_pool(psum.reshape(2 * B // 8, 128), psum.reshape(2 * B // 8, 128),
                  pcnt.reshape(2 * B // 8, 128), pcnt.reshape(2 * B // 8, 128),
                  bdl, btl)
    return zv.reshape(B, 16)[:, :1]


# speedup vs baseline: 1.2399x; 1.0076x over previous
import jax
import jax.numpy as jnp
from jax import lax
from jax.experimental import pallas as pl
from jax.experimental.pallas import tpu as pltpu
from jax.experimental.pallas import tpu_sc as plsc

N = 100000
E = 3200000
B = 256

SB = 125
NROW = E // SB
RPW = NROW // 32
CH = 4
HB = 8
NB = RPW // (2 * HB)
CB = 16
NBC = RPW // (2 * CB)

NP = 102400
RSUB = NP // 16

PB = 100
PR = NP // PB
PRW = PR // 32
PT = 384

GV = NP // 8
RV = 2 * NP // 8
BR = 1600
NBLK = GV // BR

import functools


@functools.lru_cache(maxsize=1)
def _sc_mesh():
    return plsc.VectorSubcoreMesh(core_axis_name="c", subcore_axis_name="s")


def _fill(ref, nrows, value):
    def body(i, _):
        ref[i, :] = jnp.full((16,), value, jnp.float32)
        return 0
    lax.fori_loop(0, nrows, body, 0)


def _zero_table(S, z_hbm, sid, rows_per_sub):
    pltpu.sync_copy(z_hbm.at[pl.ds(sid * rows_per_sub, rows_per_sub)],
                    S.at[pl.ds(sid * rows_per_sub, rows_per_sub)])


def _count_body(z_hbm, dst_hbm, out_hbm, idst_a, idst_b, ones_buf, S,
                sem_s, sem_ia, sem_ib):
    cid = lax.axis_index("c")
    sid = lax.axis_index("s")
    _fill(ones_buf, SB, 1.0)
    _zero_table(S, z_hbm, sid, RSUB)
    plsc.subcore_barrier()
    base = cid * (16 * RPW) + sid * RPW
    pltpu.async_copy(dst_hbm.at[pl.ds(base, CB)], idst_a, sem_ia)
    pltpu.async_copy(dst_hbm.at[pl.ds(base + CB, CB)], idst_b, sem_ib)

    def it_body(k, _):
        rn = jnp.minimum(base + (k + 1) * 2 * CB, NROW - 2 * CB)
        pltpu.make_async_copy(dst_hbm.at[pl.ds(0, CB)], idst_a, sem_ia).wait()
        da = [pltpu.async_copy(ones_buf, S.at[idst_a.at[j]], sem_s, add=True)
              for j in range(CB)]
        pltpu.make_async_copy(dst_hbm.at[pl.ds(0, CB)], idst_b, sem_ib).wait()
        db = [pltpu.async_copy(ones_buf, S.at[idst_b.at[j]], sem_s, add=True)
              for j in range(CB)]
        for d in da:
            d.wait()
        pltpu.async_copy(dst_hbm.at[pl.ds(rn, CB)], idst_a, sem_ia)
        for d in db:
            d.wait()
        rn2 = jnp.minimum(rn + CB, NROW - CB)
        pltpu.async_copy(dst_hbm.at[pl.ds(rn2, CB)], idst_b, sem_ib)
        return 0
    lax.fori_loop(0, NBC, it_body, 0)
    pltpu.make_async_copy(dst_hbm.at[pl.ds(0, CB)], idst_a, sem_ia).wait()
    pltpu.make_async_copy(dst_hbm.at[pl.ds(0, CB)], idst_b, sem_ib).wait()
    plsc.subcore_barrier()
    pltpu.sync_copy(S.at[pl.ds(sid * RSUB, RSUB)],
                    out_hbm.at[pl.ds(cid * NP + sid * RSUB, RSUB)])


@functools.lru_cache(maxsize=1)
def _sc_count():
    return pl.kernel(
        _count_body,
        out_type=jax.ShapeDtypeStruct((2 * NP, 16), jnp.float32),
        mesh=_sc_mesh(),
        compiler_params=pltpu.CompilerParams(use_tc_tiling_on_sc=False),
        scratch_types=[
            pltpu.VMEM((CB, SB), jnp.int32),
            pltpu.VMEM((CB, SB), jnp.int32),
            pltpu.VMEM((SB, 16), jnp.float32),
            pltpu.VMEM_SHARED((NP, 16), jnp.float32),
            pltpu.SemaphoreType.DMA,
            pltpu.SemaphoreType.DMA,
            pltpu.SemaphoreType.DMA,
        ],
    )


def _layer_body(z_hbm, g_hbm, src_hbm, dst_hbm, out_hbm,
                isrc_a, idst_a, isrc_b, idst_b, rows_buf, S,
                sem_g, sem_s, sem_ia, sem_ib):
    cid = lax.axis_index("c")
    sid = lax.axis_index("s")
    _zero_table(S, z_hbm, sid, RSUB)
    plsc.subcore_barrier()
    base = cid * (16 * RPW) + sid * RPW
    pltpu.async_copy(src_hbm.at[pl.ds(base, HB)], isrc_a, sem_ia)
    pltpu.async_copy(dst_hbm.at[pl.ds(base, HB)], idst_a, sem_ia)
    pltpu.async_copy(src_hbm.at[pl.ds(base + HB, HB)], isrc_b, sem_ib)
    pltpu.async_copy(dst_hbm.at[pl.ds(base + HB, HB)], idst_b, sem_ib)

    def _gfire(isrc, g0, rb):
        return [pltpu.async_copy(g_hbm.at[isrc.at[CH * g0 + j]],
                                 rows_buf.at[rb, j], sem_g)
                for j in range(CH)]

    def _sfire(idst, g0, rb):
        return [pltpu.async_copy(rows_buf.at[rb, j],
                                 S.at[idst.at[CH * g0 + j]], sem_s, add=True)
                for j in range(CH)]

    def _waitall(descs):
        for d in descs:
            d.wait()

    def it_body(k, _):
        rn = jnp.minimum(base + (k + 1) * 2 * HB, NROW - 2 * HB)
        pltpu.make_async_copy(src_hbm.at[pl.ds(0, HB)], isrc_a, sem_ia).wait()
        pltpu.make_async_copy(src_hbm.at[pl.ds(0, HB)], idst_a, sem_ia).wait()
        g0 = _gfire(isrc_a, 0, 0)
        g1 = _gfire(isrc_a, 1, 1)
        pltpu.make_async_copy(src_hbm.at[pl.ds(0, HB)], isrc_b, sem_ib).wait()
        pltpu.make_async_copy(src_hbm.at[pl.ds(0, HB)], idst_b, sem_ib).wait()
        g2 = _gfire(isrc_b, 0, 2)
        _waitall(g0)
        s0 = _sfire(idst_a, 0, 0)
        _waitall(g1)
        s1 = _sfire(idst_a, 1, 1)
        _waitall(s0)
        g3 = _gfire(isrc_b, 1, 0)
        _waitall(g2)
        s2 = _sfire(idst_b, 0, 2)
        _waitall(s1)
        pltpu.async_copy(src_hbm.at[pl.ds(rn, HB)], isrc_a, sem_ia)
        pltpu.async_copy(dst_hbm.at[pl.ds(rn, HB)], idst_a, sem_ia)
        _waitall(g3)
        s3 = _sfire(idst_b, 1, 0)
        _waitall(s2)
        _waitall(s3)
        rn2 = jnp.minimum(rn + HB, NROW - HB)
        pltpu.async_copy(src_hbm.at[pl.ds(rn2, HB)], isrc_b, sem_ib)
        pltpu.async_copy(dst_hbm.at[pl.ds(rn2, HB)], idst_b, sem_ib)
        return 0
    lax.fori_loop(0, NB, it_body, 0)
    pltpu.make_async_copy(src_hbm.at[pl.ds(0, HB)], isrc_a, sem_ia).wait()
    pltpu.make_async_copy(src_hbm.at[pl.ds(0, HB)], idst_a, sem_ia).wait()
    pltpu.make_async_copy(src_hbm.at[pl.ds(0, HB)], isrc_b, sem_ib).wait()
    pltpu.make_async_copy(src_hbm.at[pl.ds(0, HB)], idst_b, sem_ib).wait()
    plsc.subcore_barrier()
    pltpu.sync_copy(S.at[pl.ds(sid * RSUB, RSUB)],
                    out_hbm.at[pl.ds(cid * NP + sid * RSUB, RSUB)])


@functools.lru_cache(maxsize=1)
def _sc_layer():
    return pl.kernel(
        _layer_body,
        out_type=jax.ShapeDtypeStruct((2 * NP, 16), jnp.float32),
        mesh=_sc_mesh(),
        compiler_params=pltpu.CompilerParams(use_tc_tiling_on_sc=False),
        scratch_types=[
            pltpu.VMEM((HB, SB), jnp.int32),
            pltpu.VMEM((HB, SB), jnp.int32),
            pltpu.VMEM((HB, SB), jnp.int32),
            pltpu.VMEM((HB, SB), jnp.int32),
            pltpu.VMEM((3, CH, SB, 16), jnp.float32),
            pltpu.VMEM_SHARED((NP, 16), jnp.float32),
            pltpu.SemaphoreType.DMA,
            pltpu.SemaphoreType.DMA,
            pltpu.SemaphoreType.DMA,
            pltpu.SemaphoreType.DMA,
        ],
    )


def _pool_body(z_hbm, h_hbm, b_hbm, sum_hbm, cnt_hbm,
               b_buf, rows_buf, ones_buf, P, C):
    cid = lax.axis_index("c")
    sid = lax.axis_index("s")
    _fill(ones_buf, PB, 1.0)
    _zero_table(P, z_hbm, sid, PT // 16)
    _zero_table(C, z_hbm, sid, PT // 16)
    plsc.subcore_barrier()
    base = cid * (16 * PRW) + sid * PRW
    pltpu.sync_copy(b_hbm.at[pl.ds(base, PRW)], b_buf)
    pltpu.sync_copy(h_hbm.at[pl.ds(base * PB, PRW * PB)], rows_buf)

    def jloop(j, _):
        pltpu.sync_copy(rows_buf.at[pl.ds(j * PB, PB)], P.at[b_buf.at[j]], add=True)
        pltpu.sync_copy(ones_buf, C.at[b_buf.at[j]], add=True)
        return 0
    lax.fori_loop(0, PRW, jloop, 0)
    plsc.subcore_barrier()
    nsub = B // 16
    pltpu.sync_copy(P.at[pl.ds(sid * nsub, nsub)],
                    sum_hbm.at[pl.ds(cid * B + sid * nsub, nsub)])
    pltpu.sync_copy(C.at[pl.ds(sid * nsub, nsub)],
                    cnt_hbm.at[pl.ds(cid * B + sid * nsub, nsub)])


@functools.lru_cache(maxsize=1)
def _sc_pool():
    return pl.kernel(
        _pool_body,
        out_type=[jax.ShapeDtypeStruct((2 * B, 16), jnp.float32),
                  jax.ShapeDtypeStruct((2 * B, 16), jnp.float32)],
        mesh=_sc_mesh(),
        compiler_params=pltpu.CompilerParams(use_tc_tiling_on_sc=False),
        scratch_types=[
            pltpu.VMEM((PRW, PB), jnp.int32),
            pltpu.VMEM((PRW * PB, 16), jnp.float32),
            pltpu.VMEM((PB, 16), jnp.float32),
            pltpu.VMEM_SHARED((PT, 16), jnp.float32),
            pltpu.VMEM_SHARED((PT, 16), jnp.float32),
        ],
    )


def _tc1_body(c0, c1, xv, bd, dinv_ref, g1_ref):
    dinv = lax.rsqrt(c0[...] + c1[...] + 1.0)
    dinv_ref[...] = dinv
    g1_ref[...] = dinv * jnp.dot(xv[...], bd[...],
                                 preferred_element_type=jnp.float32)


_tc1 = pl.pallas_call(
    _tc1_body,
    grid=(NBLK,),
    in_specs=[pl.BlockSpec((BR, 128), lambda i: (i, 0)),
              pl.BlockSpec((BR, 128), lambda i: (i + NBLK, 0)),
              pl.BlockSpec((BR, 128), lambda i: (i, 0)),
              pl.BlockSpec((128, 128), lambda i: (0, 0))],
    out_specs=[pl.BlockSpec((BR, 128), lambda i: (i, 0)),
               pl.BlockSpec((BR, 128), lambda i: (i, 0))],
    out_shape=[jax.ShapeDtypeStruct((GV, 128), jnp.float32),
               jax.ShapeDtypeStruct((GV, 128), jnp.float32)],
)


def _tcmid_body(s0, s1, g, dinv, bt, bd, gout):
    h = dinv[...] * (s0[...] + s1[...] + g[...]) + bt[...]
    gout[...] = dinv[...] * jnp.dot(h, bd[...],
                                    preferred_element_type=jnp.float32)


_tc_mid = pl.pallas_call(
    _tcmid_body,
    grid=(NBLK,),
    in_specs=[pl.BlockSpec((BR, 128), lambda i: (i, 0)),
              pl.BlockSpec((BR, 128), lambda i: (i + NBLK, 0)),
              pl.BlockSpec((BR, 128), lambda i: (i, 0)),
              pl.BlockSpec((BR, 128), lambda i: (i, 0)),
              pl.BlockSpec((1, 128), lambda i: (0, 0)),
              pl.BlockSpec((128, 128), lambda i: (0, 0))],
    out_specs=pl.BlockSpec((BR, 128), lambda i: (i, 0)),
    out_shape=jax.ShapeDtypeStruct((GV, 128), jnp.float32),
)


def _tclast_body(s0, s1, g, dinv, bt, hout):
    hout[...] = dinv[...] * (s0[...] + s1[...] + g[...]) + bt[...]


_tc_last = pl.pallas_call(
    _tclast_body,
    grid=(NBLK,),
    in_specs=[pl.BlockSpec((BR, 128), lambda i: (i, 0)),
              pl.BlockSpec((BR, 128), lambda i: (i + NBLK, 0)),
              pl.BlockSpec((BR, 128), lambda i: (i, 0)),
              pl.BlockSpec((BR, 128), lambda i: (i, 0)),
              pl.BlockSpec((1, 128), lambda i: (0, 0))],
    out_specs=pl.BlockSpec((BR, 128), lambda i: (i, 0)),
    out_shape=jax.ShapeDtypeStruct((GV, 128), jnp.float32),
)


def _tcpool_body(p0, p1, c0, c1, wl, blt, z):
    pooled = (p0[...] + p1[...]) / jnp.maximum(c0[...] + c1[...], 1.0)
    z[...] = jax.nn.sigmoid(
        jnp.dot(pooled, wl[...], preferred_element_type=jnp.float32) + blt[...])


_tc_pool = pl.pallas_call(
    _tcpool_body,
    grid=(1,),
    in_specs=[pl.BlockSpec((B // 8, 128), lambda i: (0, 0)),
              pl.BlockSpec((B // 8, 128), lambda i: (1, 0)),
              pl.BlockSpec((B // 8, 128), lambda i: (0, 0)),
              pl.BlockSpec((B // 8, 128), lambda i: (1, 0)),
              pl.BlockSpec((128, 128), lambda i: (0, 0)),
              pl.BlockSpec((1, 128), lambda i: (0, 0))],
    out_specs=pl.BlockSpec((B // 8, 128), lambda i: (0, 0)),
    out_shape=jax.ShapeDtypeStruct((B // 8, 128), jnp.float32),
)


def kernel(x, edge_index, batch, W1, b1, W2, b2, W3, b3, Wl, bl):
    f32 = jnp.float32
    src2d = edge_index[0].reshape(NROW, SB)
    dst2d = edge_index[1].reshape(NROW, SB)
    batch2d = jnp.pad(batch, (0, NP - N), constant_values=B).reshape(PR, PB)
    xpad = jnp.pad(x, ((0, NP - N), (0, 12)))
    eye8 = jnp.eye(8, dtype=f32)
    bd1 = jnp.kron(eye8, jnp.pad(W1, ((0, 12), (0, 0))))
    bd2 = jnp.kron(eye8, W2)
    bd3 = jnp.kron(eye8, W3)
    bdl = jnp.kron(eye8, jnp.pad(Wl, ((0, 0), (0, 15))))
    bt1 = jnp.tile(b1, 8).reshape(1, 128)
    bt2 = jnp.tile(b2, 8).reshape(1, 128)
    bt3 = jnp.tile(b3, 8).reshape(1, 128)
    btl = jnp.tile(jnp.pad(bl, (0, 15)), 8).reshape(1, 128)

    zeros = jnp.zeros((NP, 16), f32)
    sc_count, sc_layer, sc_pool = _sc_count(), _sc_layer(), _sc_pool()
    cnt = sc_count(zeros, dst2d).reshape(RV, 128)
    dinv_v, g1v = _tc1(cnt, cnt, xpad.reshape(GV, 128), bd1)
    s1 = sc_layer(zeros, g1v.reshape(NP, 16), src2d, dst2d).reshape(RV, 128)
    g2v = _tc_mid(s1, s1, g1v, dinv_v, bt1, bd2)
    s2 = sc_layer(zeros, g2v.reshape(NP, 16), src2d, dst2d).reshape(RV, 128)
    g3v = _tc_mid(s2, s2, g2v, dinv_v, bt2, bd3)
    s3 = sc_layer(zeros, g3v.reshape(NP, 16), src2d, dst2d).reshape(RV, 128)
    h3v = _tc_last(s3, s3, g3v, dinv_v, bt3)
    psum, pcnt = sc_pool(zeros, h3v.reshape(NP, 16), batch2d)
    zv = _tc_pool(psum.reshape(2 * B // 8, 128), psum.reshape(2 * B // 8, 128),
                  pcnt.reshape(2 * B // 8, 128), pcnt.reshape(2 * B // 8, 128),
                  bdl, btl)
    return zv.reshape(B, 16)[:, :1]
